# baseline (speedup 1.0000x reference)
import functools

import jax
import jax.numpy as jnp
from jax import lax
from jax.experimental import pallas as pl
from jax.experimental.pallas import tpu as pltpu
from jax.experimental.pallas import tpu_sc as plsc

NS = 16
NC = 2
WIN = 128
NPAD = 16

_HI = lax.Precision.HIGHEST


def _dot(a, b):
    return jnp.dot(a, b, preferred_element_type=jnp.float32, precision=_HI)


def _mesh():
    return plsc.VectorSubcoreMesh(core_axis_name="c", subcore_axis_name="s")


def _split8(n):
    rps = -(-n // (NS * 8)) * 8
    return rps, n - (NS - 1) * rps


def _chunked(s, n, fn):
    rps, last = _split8(n)

    @pl.when(s < NS - 1)
    def _():
        fn(s * rps, rps)

    @pl.when(s == NS - 1)
    def _():
        fn((NS - 1) * rps, last)


@functools.lru_cache(None)
def _sc_degree(n, e_pad):
    epw = e_pad // NS
    nwin = epw // WIN
    rz, _ = _split8(n + NPAD)

    @functools.partial(
        pl.kernel, mesh=_mesh(),
        out_type=jax.ShapeDtypeStruct((2 * n, 128), jnp.float32),
        scratch_types=[
            pltpu.VMEM_SHARED((n + NPAD, 128), jnp.float32),
            pltpu.VMEM((nwin, WIN), jnp.int32),
            pltpu.VMEM((WIN, 128), jnp.float32),
            pltpu.SemaphoreType.DMA,
        ])
    def deg_kernel(dst_hbm, ones_hbm, zeros_hbm, deg_hbm, acc, dst_loc, ones_v, sem):
        c = lax.axis_index("c")
        s = lax.axis_index("s")
        row0 = (c * NS + s) * nwin
        pltpu.sync_copy(dst_hbm.at[pl.ds(row0, nwin)], dst_loc)
        _chunked(s, n + NPAD,
                 lambda off, sz: pltpu.sync_copy(zeros_hbm.at[pl.ds(0, sz)],
                                                 acc.at[pl.ds(off, sz)]))
        pltpu.sync_copy(ones_hbm, ones_v)
        plsc.subcore_barrier()

        @pl.loop(0, nwin)
        def _(w):
            pltpu.sync_copy(ones_v, acc.at[dst_loc.at[w]], add=True)

        plsc.subcore_barrier()
        _chunked(s, n,
                 lambda off, sz: pltpu.sync_copy(
                     acc.at[pl.ds(off, sz)],
                     deg_hbm.at[pl.ds(c * n + off, sz)]))

    return deg_kernel


@functools.lru_cache(None)
def _sc_agg(n, e_pad, f):
    epw = e_pad // NS
    nwin = epw // WIN

    blk = min(32, nwin)
    nblk = nwin // blk

    @functools.partial(
        pl.kernel, mesh=_mesh(),
        out_type=jax.ShapeDtypeStruct((2 * n, f), jnp.float32),
        scratch_types=[
            pltpu.VMEM_SHARED((n + NPAD, f), jnp.float32),
            pltpu.VMEM((blk, WIN), jnp.int32),
            pltpu.VMEM((blk, WIN), jnp.int32),
            pltpu.VMEM((WIN, f), jnp.float32),
            pltpu.VMEM((WIN, f), jnp.float32),
            pltpu.SemaphoreType.DMA,
            pltpu.SemaphoreType.DMA,
        ])
    def agg_kernel(y_hbm, src_hbm, dst_hbm, out_hbm, acc,
                   src_loc, dst_loc, rows0, rows1, sem0, sem1):
        c = lax.axis_index("c")
        s = lax.axis_index("s")
        row0 = (c * NS + s) * nwin
        _chunked(s, n,
                 lambda off, sz: pltpu.sync_copy(
                     y_hbm.at[pl.ds(c * n + off, sz)],
                     acc.at[pl.ds(off, sz)]))
        plsc.subcore_barrier()

        @pl.loop(0, nblk)
        def _(bi):
            pltpu.sync_copy(src_hbm.at[pl.ds(row0 + bi * blk, blk)], src_loc)
            pltpu.sync_copy(dst_hbm.at[pl.ds(row0 + bi * blk, blk)], dst_loc)

            @pl.loop(0, blk)
            def _(w):
                pltpu.async_copy(y_hbm.at[src_loc.at[w]], rows0, sem0).wait()
                pltpu.sync_copy(rows0, acc.at[dst_loc.at[w]], add=True)

        plsc.subcore_barrier()
        _chunked(s, n,
                 lambda off, sz: pltpu.sync_copy(
                     acc.at[pl.ds(off, sz)],
                     out_hbm.at[pl.ds(c * n + off, sz)]))

    return agg_kernel


def _full(shape):
    return pl.BlockSpec(shape, lambda i: (0, 0))


def _rows(r, k):
    return pl.BlockSpec((r, k), lambda i: (i, 0))


def _row_block(m):
    for r in (1000, 2000, 504, 8):
        if m % r == 0:
            return r
    return m


@functools.lru_cache(None)
def _tc_pre(m):
    r = _row_block(m)

    def body(x_ref, f_ref, w0, b0, w1, b1, w2, b2, out_ref):
        nb = jnp.maximum(_dot(x_ref[...], w0[...]) + b0[...], 0.0)
        pb = jnp.maximum(_dot(f_ref[...], w1[...]) + b1[...], 0.0)
        pb = jnp.maximum(_dot(pb, w2[...]) + b2[...], 0.0)
        out_ref[...] = jnp.concatenate([nb, pb], axis=1)

    return pl.pallas_call(
        body,
        grid=(m // r,),
        in_specs=[_rows(r, 3), _rows(r, 128), _full((3, 128)), _full((1, 128)),
                  _full((128, 256)), _full((1, 256)), _full((256, 128)),
                  _full((1, 128))],
        out_specs=_rows(r, 256),
        out_shape=jax.ShapeDtypeStruct((m, 256), jnp.float32),
    )


@functools.lru_cache(None)
def _tc_y1(m):
    r = _row_block(m)

    def body(ft, dg, w, y_ref, dinv_ref):
        dinv = jnp.broadcast_to(lax.rsqrt(dg[...][:, 0:1] + 1.0), (r, 128))
        y_ref[...] = _dot(ft[...], w[...]) * dinv
        dinv_ref[...] = dinv

    return pl.pallas_call(
        body,
        grid=(m // r,),
        in_specs=[_rows(r, 256), _rows(r, 128), _full((256, 128))],
        out_specs=[_rows(r, 128), _rows(r, 128)],
        out_shape=[jax.ShapeDtypeStruct((m, 128), jnp.float32),
                   jax.ShapeDtypeStruct((m, 128), jnp.float32)],
    )


@functools.lru_cache(None)
def _tc_post(m, hp_width, emit_y):
    r = _row_block(m)

    def body(pa, dv, hp, gb, fcw, fcb, pjw, pjb, *rest):
        t = jnp.maximum(pa[...] * dv[...] + gb[...], 0.0)
        out = _dot(t, fcw[...]) + fcb[...]
        cat = jnp.concatenate([out, hp[...]], axis=1)
        h = jnp.maximum(_dot(cat, pjw[...]) + pjb[...] + out, 0.0)
        if emit_y:
            nw, h_ref, y_ref = rest
            h_ref[...] = h
            y_ref[...] = _dot(h, nw[...]) * dv[...]
        else:
            rest[0][...] = h

    in_specs = [_rows(r, 128), _rows(r, 128), _rows(r, hp_width),
                _full((1, 128)), _full((128, 64)), _full((1, 64)),
                _full((64 + hp_width, 64)), _full((1, 64))]
    out_specs = [_rows(r, 64)]
    out_shape = [jax.ShapeDtypeStruct((m, 64), jnp.float32)]
    if emit_y:
        in_specs.append(_full((64, 128)))
        out_specs.append(_rows(r, 128))
        out_shape.append(jax.ShapeDtypeStruct((m, 128), jnp.float32))
    return pl.pallas_call(
        body,
        grid=(m // r,),
        in_specs=in_specs,
        out_specs=out_specs,
        out_shape=out_shape,
    )


@functools.lru_cache(None)
def _tc_glob(m):
    def body(g, w1, b1, w2, b2, wp, bp, out_ref):
        t = jnp.maximum(_dot(g[...], w1[...]) + b1[...], 0.0)
        t = jnp.maximum(_dot(t, w2[...]) + b2[...], 0.0)
        out_ref[...] = _dot(t, wp[...]) + bp[...]

    return pl.pallas_call(
        body,
        grid=(1,),
        in_specs=[_rows(m, 128), _full((128, 256)), _full((1, 256)),
                  _full((256, 128)), _full((1, 128)), _full((128, 64)),
                  _full((1, 64))],
        out_specs=_rows(m, 64),
        out_shape=jax.ShapeDtypeStruct((m, 64), jnp.float32),
    )


def kernel(g1_x, g1_f, g1_g, g1_edge_index, g2_x, g2_f, g2_g, g2_edge_index,
           params):
    p = params
    n = g1_x.shape[0]
    e = g1_edge_index.shape[1]
    ng = g1_g.shape[0]
    m = 2 * n
    i32 = jnp.int32

    chunk = NS * WIN * 8
    e_pad = ((e + chunk - 1) // chunk) * chunk
    pad = e_pad - e

    X = jnp.concatenate([g1_x, g2_x], axis=0)
    F0 = jnp.concatenate([g1_f, g2_f], axis=0)
    G = jnp.concatenate([g1_g, g2_g], axis=0)

    pad_dst = n + (jnp.arange(pad, dtype=i32) % NPAD)
    pad_src = jnp.zeros((pad,), i32)
    src_flat = jnp.concatenate([g1_edge_index[0], pad_src,
                                g2_edge_index[0] + n, pad_src + n])
    dst_flat = jnp.concatenate([g1_edge_index[1], pad_dst,
                                g2_edge_index[1], pad_dst])
    src_flat = src_flat.reshape(-1, WIN)
    dst_flat = dst_flat.reshape(-1, WIN)

    ones16 = jnp.ones((WIN, 128), jnp.float32)
    zeros16 = jnp.zeros((_split8(n + NPAD)[0], 128), jnp.float32)

    b = {k: p[k].reshape(1, -1) for k in
         ('node_lin_b', 'patch1_b', 'patch2_b', 'glob1_b', 'glob2_b',
          'gcn1_b', 'fc1_b', 'gcn2_b', 'fc2_b', 'gcn3_b', 'fc3_b',
          'gcn4_b', 'fc4_b', 'proj1_b', 'proj2_b', 'proj3_b', 'proj4_b',
          'gproj_b')}

    deg16 = _sc_degree(n, e_pad)(dst_flat, ones16, zeros16)
    feats = _tc_pre(m)(X, F0, p['node_lin_W'], b['node_lin_b'],
                       p['patch1_W'], b['patch1_b'],
                       p['patch2_W'], b['patch2_b'])
    y1, dinv = _tc_y1(m)(feats, deg16, p['gcn1_W'])

    agg = _sc_agg(n, e_pad, 128)
    p1 = agg(y1, src_flat, dst_flat)
    h1, y2 = _tc_post(m, 256, True)(p1, dinv, feats, b['gcn1_b'],
                                    p['fc1_W'], b['fc1_b'],
                                    p['proj1_W'], b['proj1_b'], p['gcn2_W'])
    p2 = agg(y2, src_flat, dst_flat)
    h2, y3 = _tc_post(m, 64, True)(p2, dinv, h1, b['gcn2_b'],
                                   p['fc2_W'], b['fc2_b'],
                                   p['proj2_W'], b['proj2_b'], p['gcn3_W'])
    p3 = agg(y3, src_flat, dst_flat)
    h3, y4 = _tc_post(m, 64, True)(p3, dinv, h2, b['gcn3_b'],
                                   p['fc3_W'], b['fc3_b'],
                                   p['proj3_W'], b['proj3_b'], p['gcn4_W'])
    p4 = agg(y4, src_flat, dst_flat)
    h4 = _tc_post(m, 64, False)(p4, dinv, h3, b['gcn4_b'],
                                p['fc4_W'], b['fc4_b'],
                                p['proj4_W'], b['proj4_b'])[0]
    gout = _tc_glob(2 * ng)(G, p['glob1_W'], b['glob1_b'],
                            p['glob2_W'], b['glob2_b'],
                            p['gproj_W'], b['gproj_b'])
    return h4[:n], h4[n:], gout[:ng], gout[ng:]


# speedup vs baseline: 1.5984x; 1.5984x over previous
import functools

import jax
import jax.numpy as jnp
from jax import lax
from jax.experimental import pallas as pl
from jax.experimental.pallas import tpu as pltpu
from jax.experimental.pallas import tpu_sc as plsc

NS = 16
NC = 2
WIN = 128
NPAD = 16

_HI = lax.Precision.HIGHEST


def _dot(a, b):
    return jnp.dot(a, b, preferred_element_type=jnp.float32, precision=_HI)


def _mesh():
    return plsc.VectorSubcoreMesh(core_axis_name="c", subcore_axis_name="s")


def _split8(n):
    rps = -(-n // (NS * 8)) * 8
    return rps, n - (NS - 1) * rps


def _chunked(s, n, fn):
    rps, last = _split8(n)

    @pl.when(s < NS - 1)
    def _():
        fn(s * rps, rps)

    @pl.when(s == NS - 1)
    def _():
        fn((NS - 1) * rps, last)


@functools.lru_cache(None)
def _sc_degree(n, e_pad):
    epw = e_pad // NS
    nwin = epw // WIN
    rz, _ = _split8(n + NPAD)

    @functools.partial(
        pl.kernel, mesh=_mesh(),
        out_type=jax.ShapeDtypeStruct((2 * n, 128), jnp.float32),
        scratch_types=[
            pltpu.VMEM_SHARED((n + NPAD, 128), jnp.float32),
            pltpu.VMEM((WIN,), jnp.int32),
            pltpu.VMEM((WIN,), jnp.int32),
            pltpu.VMEM((WIN, 128), jnp.float32),
            pltpu.SemaphoreType.DMA,
            pltpu.SemaphoreType.DMA,
        ])
    def deg_kernel(dst_hbm, ones_hbm, zeros_hbm, deg_hbm, acc,
                   dv0, dv1, ones_v, si0, si1):
        c = lax.axis_index("c")
        s = lax.axis_index("s")
        base0 = c * e_pad + s * epw

        def idx_start(w, dv, sem):
            pltpu.async_copy(dst_hbm.at[pl.ds(base0 + w * WIN, WIN)], dv, sem)

        def idx_wait(w, dv, sem):
            pltpu.make_async_copy(dst_hbm.at[pl.ds(base0 + w * WIN, WIN)],
                                  dv, sem).wait()

        _chunked(s, n + NPAD,
                 lambda off, sz: pltpu.sync_copy(zeros_hbm.at[pl.ds(0, sz)],
                                                 acc.at[pl.ds(off, sz)]))
        pltpu.sync_copy(ones_hbm, ones_v)
        plsc.subcore_barrier()

        idx_start(0, dv0, si0)

        @pl.loop(0, nwin, step=2)
        def _(w):
            idx_wait(w, dv0, si0)
            idx_start(w + 1, dv1, si1)
            pltpu.sync_copy(ones_v, acc.at[dv0], add=True)
            idx_wait(w + 1, dv1, si1)

            @pl.when(w + 2 < nwin)
            def _():
                idx_start(w + 2, dv0, si0)

            pltpu.sync_copy(ones_v, acc.at[dv1], add=True)

        plsc.subcore_barrier()
        _chunked(s, n,
                 lambda off, sz: pltpu.sync_copy(
                     acc.at[pl.ds(off, sz)],
                     deg_hbm.at[pl.ds(c * n + off, sz)]))

    return deg_kernel


@functools.lru_cache(None)
def _sc_agg(n, e_pad, f):
    epw = e_pad // NS
    nwin = epw // WIN

    @functools.partial(
        pl.kernel, mesh=_mesh(),
        out_type=jax.ShapeDtypeStruct((2 * n, f), jnp.float32),
        scratch_types=[
            pltpu.VMEM_SHARED((n + NPAD, f), jnp.float32),
            pltpu.VMEM((WIN,), jnp.int32),
            pltpu.VMEM((WIN,), jnp.int32),
            pltpu.VMEM((WIN,), jnp.int32),
            pltpu.VMEM((WIN,), jnp.int32),
            pltpu.VMEM((WIN, f), jnp.float32),
            pltpu.VMEM((WIN, f), jnp.float32),
            pltpu.SemaphoreType.DMA,
            pltpu.SemaphoreType.DMA,
            pltpu.SemaphoreType.DMA,
            pltpu.SemaphoreType.DMA,
        ])
    def agg_kernel(y_hbm, src_hbm, dst_hbm, out_hbm, acc,
                   sv0, dv0, sv1, dv1, rows0, rows1, si0, si1, sg0, sg1):
        c = lax.axis_index("c")
        s = lax.axis_index("s")
        base0 = c * e_pad + s * epw

        def idx_start(w, sv, dv, sem):
            pltpu.async_copy(src_hbm.at[pl.ds(base0 + w * WIN, WIN)], sv, sem)
            pltpu.async_copy(dst_hbm.at[pl.ds(base0 + w * WIN, WIN)], dv, sem)

        def idx_wait(w, sv, dv, sem):
            pltpu.make_async_copy(src_hbm.at[pl.ds(base0 + w * WIN, WIN)],
                                  sv, sem).wait()
            pltpu.make_async_copy(dst_hbm.at[pl.ds(base0 + w * WIN, WIN)],
                                  dv, sem).wait()

        _chunked(s, n,
                 lambda off, sz: pltpu.sync_copy(
                     y_hbm.at[pl.ds(c * n + off, sz)],
                     acc.at[pl.ds(off, sz)]))
        plsc.subcore_barrier()

        idx_start(0, sv0, dv0, si0)
        idx_wait(0, sv0, dv0, si0)
        idx_start(1, sv1, dv1, si1)
        pltpu.async_copy(y_hbm.at[sv0], rows0, sg0)

        @pl.loop(0, nwin, step=2)
        def _(w):
            pltpu.make_async_copy(y_hbm.at[sv0], rows0, sg0).wait()
            idx_wait(w + 1, sv1, dv1, si1)
            pltpu.async_copy(y_hbm.at[sv1], rows1, sg1)
            pltpu.sync_copy(rows0, acc.at[dv0], add=True)

            @pl.when(w + 2 < nwin)
            def _():
                idx_start(w + 2, sv0, dv0, si0)

            pltpu.make_async_copy(y_hbm.at[sv1], rows1, sg1).wait()

            @pl.when(w + 2 < nwin)
            def _():
                idx_wait(w + 2, sv0, dv0, si0)
                pltpu.async_copy(y_hbm.at[sv0], rows0, sg0)

            pltpu.sync_copy(rows1, acc.at[dv1], add=True)

            @pl.when(w + 3 < nwin)
            def _():
                idx_start(w + 3, sv1, dv1, si1)

        plsc.subcore_barrier()
        _chunked(s, n,
                 lambda off, sz: pltpu.sync_copy(
                     acc.at[pl.ds(off, sz)],
                     out_hbm.at[pl.ds(c * n + off, sz)]))

    return agg_kernel


def _full(shape):
    return pl.BlockSpec(shape, lambda i: (0, 0))


def _rows(r, k):
    return pl.BlockSpec((r, k), lambda i: (i, 0))


def _row_block(m):
    for r in (1000, 2000, 504, 8):
        if m % r == 0:
            return r
    return m


@functools.lru_cache(None)
def _tc_pre(m):
    r = _row_block(m)

    def body(x_ref, f_ref, w0, b0, w1, b1, w2, b2, out_ref):
        nb = jnp.maximum(_dot(x_ref[...], w0[...]) + b0[...], 0.0)
        pb = jnp.maximum(_dot(f_ref[...], w1[...]) + b1[...], 0.0)
        pb = jnp.maximum(_dot(pb, w2[...]) + b2[...], 0.0)
        out_ref[...] = jnp.concatenate([nb, pb], axis=1)

    return pl.pallas_call(
        body,
        grid=(m // r,),
        in_specs=[_rows(r, 3), _rows(r, 128), _full((3, 128)), _full((1, 128)),
                  _full((128, 256)), _full((1, 256)), _full((256, 128)),
                  _full((1, 128))],
        out_specs=_rows(r, 256),
        out_shape=jax.ShapeDtypeStruct((m, 256), jnp.float32),
    )


@functools.lru_cache(None)
def _tc_y1(m):
    r = _row_block(m)

    def body(ft, dg, w, y_ref, dinv_ref):
        dinv = jnp.broadcast_to(lax.rsqrt(dg[...][:, 0:1] + 1.0), (r, 128))
        y_ref[...] = _dot(ft[...], w[...]) * dinv
        dinv_ref[...] = dinv

    return pl.pallas_call(
        body,
        grid=(m // r,),
        in_specs=[_rows(r, 256), _rows(r, 128), _full((256, 128))],
        out_specs=[_rows(r, 128), _rows(r, 128)],
        out_shape=[jax.ShapeDtypeStruct((m, 128), jnp.float32),
                   jax.ShapeDtypeStruct((m, 128), jnp.float32)],
    )


@functools.lru_cache(None)
def _tc_post(m, hp_width, emit_y):
    r = _row_block(m)

    def body(pa, dv, hp, gb, fcw, fcb, pjw, pjb, *rest):
        t = jnp.maximum(pa[...] * dv[...] + gb[...], 0.0)
        out = _dot(t, fcw[...]) + fcb[...]
        cat = jnp.concatenate([out, hp[...]], axis=1)
        h = jnp.maximum(_dot(cat, pjw[...]) + pjb[...] + out, 0.0)
        if emit_y:
            nw, h_ref, y_ref = rest
            h_ref[...] = h
            y_ref[...] = _dot(h, nw[...]) * dv[...]
        else:
            rest[0][...] = h

    in_specs = [_rows(r, 128), _rows(r, 128), _rows(r, hp_width),
                _full((1, 128)), _full((128, 64)), _full((1, 64)),
                _full((64 + hp_width, 64)), _full((1, 64))]
    out_specs = [_rows(r, 64)]
    out_shape = [jax.ShapeDtypeStruct((m, 64), jnp.float32)]
    if emit_y:
        in_specs.append(_full((64, 128)))
        out_specs.append(_rows(r, 128))
        out_shape.append(jax.ShapeDtypeStruct((m, 128), jnp.float32))
    return pl.pallas_call(
        body,
        grid=(m // r,),
        in_specs=in_specs,
        out_specs=out_specs,
        out_shape=out_shape,
    )


@functools.lru_cache(None)
def _tc_glob(m):
    def body(g, w1, b1, w2, b2, wp, bp, out_ref):
        t = jnp.maximum(_dot(g[...], w1[...]) + b1[...], 0.0)
        t = jnp.maximum(_dot(t, w2[...]) + b2[...], 0.0)
        out_ref[...] = _dot(t, wp[...]) + bp[...]

    return pl.pallas_call(
        body,
        grid=(1,),
        in_specs=[_rows(m, 128), _full((128, 256)), _full((1, 256)),
                  _full((256, 128)), _full((1, 128)), _full((128, 64)),
                  _full((1, 64))],
        out_specs=_rows(m, 64),
        out_shape=jax.ShapeDtypeStruct((m, 64), jnp.float32),
    )


def kernel(g1_x, g1_f, g1_g, g1_edge_index, g2_x, g2_f, g2_g, g2_edge_index,
           params):
    p = params
    n = g1_x.shape[0]
    e = g1_edge_index.shape[1]
    ng = g1_g.shape[0]
    m = 2 * n
    i32 = jnp.int32

    chunk = NS * WIN * 2
    e_pad = ((e + chunk - 1) // chunk) * chunk
    pad = e_pad - e

    X = jnp.concatenate([g1_x, g2_x], axis=0)
    F0 = jnp.concatenate([g1_f, g2_f], axis=0)
    G = jnp.concatenate([g1_g, g2_g], axis=0)

    pad_dst = n + (jnp.arange(pad, dtype=i32) % NPAD)
    pad_src = jnp.zeros((pad,), i32)
    src_flat = jnp.concatenate([g1_edge_index[0], pad_src,
                                g2_edge_index[0] + n, pad_src + n])
    dst_flat = jnp.concatenate([g1_edge_index[1], pad_dst,
                                g2_edge_index[1], pad_dst])

    ones16 = jnp.ones((WIN, 128), jnp.float32)
    zeros16 = jnp.zeros((_split8(n + NPAD)[0], 128), jnp.float32)

    b = {k: p[k].reshape(1, -1) for k in
         ('node_lin_b', 'patch1_b', 'patch2_b', 'glob1_b', 'glob2_b',
          'gcn1_b', 'fc1_b', 'gcn2_b', 'fc2_b', 'gcn3_b', 'fc3_b',
          'gcn4_b', 'fc4_b', 'proj1_b', 'proj2_b', 'proj3_b', 'proj4_b',
          'gproj_b')}

    deg16 = _sc_degree(n, e_pad)(dst_flat, ones16, zeros16)
    feats = _tc_pre(m)(X, F0, p['node_lin_W'], b['node_lin_b'],
                       p['patch1_W'], b['patch1_b'],
                       p['patch2_W'], b['patch2_b'])
    y1, dinv = _tc_y1(m)(feats, deg16, p['gcn1_W'])

    agg = _sc_agg(n, e_pad, 128)
    p1 = agg(y1, src_flat, dst_flat)
    h1, y2 = _tc_post(m, 256, True)(p1, dinv, feats, b['gcn1_b'],
                                    p['fc1_W'], b['fc1_b'],
                                    p['proj1_W'], b['proj1_b'], p['gcn2_W'])
    p2 = agg(y2, src_flat, dst_flat)
    h2, y3 = _tc_post(m, 64, True)(p2, dinv, h1, b['gcn2_b'],
                                   p['fc2_W'], b['fc2_b'],
                                   p['proj2_W'], b['proj2_b'], p['gcn3_W'])
    p3 = agg(y3, src_flat, dst_flat)
    h3, y4 = _tc_post(m, 64, True)(p3, dinv, h2, b['gcn3_b'],
                                   p['fc3_W'], b['fc3_b'],
                                   p['proj3_W'], b['proj3_b'], p['gcn4_W'])
    p4 = agg(y4, src_flat, dst_flat)
    h4 = _tc_post(m, 64, False)(p4, dinv, h3, b['gcn4_b'],
                                p['fc4_W'], b['fc4_b'],
                                p['proj4_W'], b['proj4_b'])[0]
    gout = _tc_glob(2 * ng)(G, p['glob1_W'], b['glob1_b'],
                            p['glob2_W'], b['glob2_b'],
                            p['gproj_W'], b['gproj_b'])
    return h4[:n], h4[n:], gout[:ng], gout[ng:]
